# baseline (speedup 1.0000x reference)
import functools

import jax
import jax.numpy as jnp
from jax import lax
from jax.experimental import pallas as pl
from jax.experimental.pallas import tpu as pltpu
from jax.experimental.pallas import tpu_sc as plsc

DIM = 768
NUM_CORES = 2
NUM_SUBCORES = 16
NW = NUM_CORES * NUM_SUBCORES

CHUNK = 16


@functools.cache
def _make_gather(total_rows: int):
    b_per_w = total_rows // NW
    n_pairs = b_per_w // (2 * CHUNK)
    mesh = plsc.VectorSubcoreMesh(core_axis_name="c", subcore_axis_name="s")

    @functools.partial(
        pl.kernel,
        mesh=mesh,
        out_type=jax.ShapeDtypeStruct((total_rows, DIM), jnp.float32),
        scratch_types=[
            pltpu.VMEM((b_per_w,), jnp.int32),
            pltpu.VMEM((CHUNK, DIM), jnp.float32),
            pltpu.VMEM((CHUNK, DIM), jnp.float32),
            pltpu.VMEM_SHARED((NUM_SUBCORES, 2, CHUNK, DIM), jnp.float32),
            pltpu.SemaphoreType.DMA,
            pltpu.SemaphoreType.DMA,
            pltpu.SemaphoreType.DMA,
            pltpu.SemaphoreType.DMA,
            pltpu.SemaphoreType.DMA,
            pltpu.SemaphoreType.DMA,
        ],
    )
    def gather_kernel(idx_hbm, table_hbm, out_hbm, idx_v, rows0, rows1,
                      spmem, gsem0, gsem1, xsem0, xsem1, dsem0, dsem1):
        sid = lax.axis_index("s")
        wid = sid * NUM_CORES + lax.axis_index("c")
        base = wid * b_per_w
        rows = (rows0, rows1)
        gsem = (gsem0, gsem1)
        xsem = (xsem0, xsem1)
        dsem = (dsem0, dsem1)

        pltpu.sync_copy(idx_hbm.at[pl.ds(base, b_per_w)], idx_v)

        def start_gather(chunk, buf):
            return pltpu.async_copy(
                table_hbm.at[idx_v.at[pl.ds(chunk * CHUNK, CHUNK)]],
                rows[buf], gsem[buf])

        def start_cross(buf):
            return pltpu.async_copy(rows[buf], spmem.at[sid, buf],
                                    xsem[buf])

        def start_drain(chunk, buf):
            return pltpu.async_copy(
                spmem.at[sid, buf],
                out_hbm.at[pl.ds(base + chunk * CHUNK, CHUNK)], dsem[buf])

        def wait_drain(buf):
            pltpu.make_async_copy(
                spmem.at[sid, buf], out_hbm.at[pl.ds(base, CHUNK)],
                dsem[buf]).wait()

        def body(g, carry):
            a = 2 * g
            ga = start_gather(a, 0)
            gb = start_gather(a + 1, 1)
            ga.wait()

            @pl.when(g > 0)
            def _():
                wait_drain(0)

            xa = start_cross(0)
            gb.wait()

            @pl.when(g > 0)
            def _():
                wait_drain(1)

            xb = start_cross(1)
            xa.wait()
            start_drain(a, 0)
            xb.wait()
            start_drain(a + 1, 1)
            return carry

        lax.fori_loop(0, n_pairs, body, 0)
        wait_drain(0)
        wait_drain(1)

    return gather_kernel


def kernel(inputs, table):
    batch, seqlen = inputs.shape
    flat_ids = inputs.reshape(-1).astype(jnp.int32)
    out = _make_gather(batch * seqlen)(flat_ids, table)
    return out.reshape(batch, seqlen, DIM)


# speedup vs baseline: 1.1355x; 1.1355x over previous
import functools

import jax
import jax.numpy as jnp
from jax import lax
from jax.experimental import pallas as pl
from jax.experimental.pallas import tpu as pltpu
from jax.experimental.pallas import tpu_sc as plsc

DIM = 768
NUM_CORES = 2
NUM_SUBCORES = 16
NW = NUM_CORES * NUM_SUBCORES

CHUNK = 16


@functools.cache
def _make_gather(total_rows: int):
    b_per_w = total_rows // NW
    n_groups = b_per_w // (4 * CHUNK)
    mesh = plsc.VectorSubcoreMesh(core_axis_name="c", subcore_axis_name="s")

    row_buf = pltpu.VMEM((CHUNK, DIM), jnp.float32)
    dma = pltpu.SemaphoreType.DMA

    @functools.partial(
        pl.kernel,
        mesh=mesh,
        out_type=jax.ShapeDtypeStruct((total_rows, DIM), jnp.float32),
        scratch_types=[
            pltpu.VMEM((b_per_w,), jnp.int32),
            row_buf, row_buf, row_buf, row_buf,
            pltpu.VMEM_SHARED((NUM_SUBCORES, 2, CHUNK, DIM), jnp.float32),
            dma, dma, dma, dma, dma, dma, dma, dma, dma, dma,
        ],
    )
    def gather_kernel(idx_hbm, table_hbm, out_hbm, idx_v,
                      rows_a0, rows_a1, rows_b0, rows_b1, spmem,
                      ga0, ga1, gb0, gb1, sa0, sa1, xb0, xb1, dp0, dp1):
        sid = lax.axis_index("s")
        wid = sid * NUM_CORES + lax.axis_index("c")
        base = wid * b_per_w
        rows_a = (rows_a0, rows_a1)
        rows_b = (rows_b0, rows_b1)
        gsem_a = (ga0, ga1)
        gsem_b = (gb0, gb1)
        ssem_a = (sa0, sa1)
        xsem_b = (xb0, xb1)
        dsem_p = (dp0, dp1)

        pltpu.sync_copy(idx_hbm.at[pl.ds(base, b_per_w)], idx_v)

        def gather(chunk, buf, sem):
            return pltpu.async_copy(
                table_hbm.at[idx_v.at[pl.ds(chunk * CHUNK, CHUNK)]],
                buf, sem)

        def out_slice(chunk):
            return out_hbm.at[pl.ds(base + chunk * CHUNK, CHUNK)]

        def drain(src, sem):
            pltpu.make_async_copy(src, out_slice(0), sem).wait()

        def body(g, carry):
            c0 = 4 * g
            nonfirst = g > 0

            @pl.when(nonfirst)
            def _():
                drain(rows_a[0], ssem_a[0])
            gd0 = gather(c0, rows_a[0], gsem_a[0])
            gs0 = gather(c0 + 1, rows_b[0], gsem_b[0])
            gd0.wait()
            pltpu.async_copy(rows_a[0], out_slice(c0), ssem_a[0])
            @pl.when(nonfirst)
            def _():
                drain(rows_a[1], ssem_a[1])
            gd1 = gather(c0 + 2, rows_a[1], gsem_a[1])
            gs0.wait()
            @pl.when(nonfirst)
            def _():
                drain(spmem.at[sid, 0], dsem_p[0])
            x0 = pltpu.async_copy(rows_b[0], spmem.at[sid, 0], xsem_b[0])
            gs1 = gather(c0 + 3, rows_b[1], gsem_b[1])
            gd1.wait()
            pltpu.async_copy(rows_a[1], out_slice(c0 + 2), ssem_a[1])
            x0.wait()
            pltpu.async_copy(spmem.at[sid, 0], out_slice(c0 + 1), dsem_p[0])
            gs1.wait()
            @pl.when(nonfirst)
            def _():
                drain(spmem.at[sid, 1], dsem_p[1])
            x1 = pltpu.async_copy(rows_b[1], spmem.at[sid, 1], xsem_b[1])
            x1.wait()
            pltpu.async_copy(spmem.at[sid, 1], out_slice(c0 + 3), dsem_p[1])
            return carry

        lax.fori_loop(0, n_groups, body, 0)
        drain(rows_a[0], ssem_a[0])
        drain(rows_a[1], ssem_a[1])
        drain(spmem.at[sid, 0], dsem_p[0])
        drain(spmem.at[sid, 1], dsem_p[1])

    return gather_kernel


def kernel(inputs, table):
    batch, seqlen = inputs.shape
    flat_ids = inputs.reshape(-1).astype(jnp.int32)
    out = _make_gather(batch * seqlen)(flat_ids, table)
    return out.reshape(batch, seqlen, DIM)
